# baseline (speedup 1.0000x reference)
import dataclasses

import jax
import jax.numpy as jnp
from jax import lax
from jax.experimental import pallas as pl
from jax.experimental.pallas import tpu as pltpu
from jax.experimental.pallas import tpu_sc as plsc

BS, NA, NH = 1024, 16, 26
VOCAB, D = 100000, 32
NUM_IDX = BS * NA * NH
LANES = 128
PACK = LANES // D

NC, NS = 2, 16
NW = NC * NS
PER_W = NUM_IDX // NW
CHUNK = 64
N_CHUNK = PER_W // CHUNK
GROUPS = CHUNK // 16


def kernel(prev_act, tables):
    wide = tables.reshape(NH * VOCAB // PACK, LANES)
    offs = jnp.arange(NH, dtype=jnp.int32) * VOCAB
    g = (prev_act.reshape(BS * NA, NH) + offs[None, :]).reshape(
        NW, PER_W // 16, 16
    )

    mesh = plsc.VectorSubcoreMesh(core_axis_name="c", subcore_axis_name="s")
    cp = pltpu.CompilerParams()
    if "needs_layout_passes" in pltpu.CompilerParams.__dataclass_fields__:
        cp = dataclasses.replace(cp, needs_layout_passes=False)

    @pl.kernel(
        out_type=jax.ShapeDtypeStruct((NUM_IDX, D), jnp.float32),
        mesh=mesh,
        compiler_params=cp,
        scratch_types=[
            pltpu.VMEM((PER_W // 16, 16), jnp.int32),
            pltpu.VMEM((1, CHUNK), jnp.int32),
            pltpu.VMEM((CHUNK, LANES), jnp.float32),
            pltpu.VMEM((CHUNK, D), jnp.float32),
        ],
    )
    def gather_kernel(wide_hbm, g_hbm, out_hbm, g_v, idx4_v, rows_v, out_v):
        wid = lax.axis_index("s") * NC + lax.axis_index("c")
        pltpu.sync_copy(g_hbm.at[wid], g_v)
        base = wid * PER_W
        iota = lax.iota(jnp.int32, 16)

        @pl.loop(0, N_CHUNK)
        def _(c):
            @pl.loop(0, GROUPS)
            def _(k):
                idx4_v[0, pl.ds(k * 16, 16)] = g_v[c * GROUPS + k] >> 2

            pltpu.sync_copy(wide_hbm.at[idx4_v.at[0]], rows_v)

            @pl.loop(0, GROUPS)
            def _(gi):
                rvec = iota + gi * 16
                s16 = (g_v[c * GROUPS + gi] & 3) * D
                for j in range(D):
                    val = plsc.load_gather(rows_v, [rvec, s16 + j])
                    plsc.store_scatter(out_v, [rvec, iota * 0 + j], val)

            pltpu.sync_copy(
                out_v, out_hbm.at[pl.ds(base + c * CHUNK, CHUNK)]
            )

    out = gather_kernel(wide, g)
    return out.reshape(BS, NA, NH * D)


# speedup vs baseline: 1.1648x; 1.1648x over previous
import dataclasses

import jax
import jax.numpy as jnp
from jax import lax
from jax.experimental import pallas as pl
from jax.experimental.pallas import tpu as pltpu
from jax.experimental.pallas import tpu_sc as plsc

BS, NA, NH = 1024, 16, 26
VOCAB, D = 100000, 32
NUM_IDX = BS * NA * NH
LANES = 128
PACK = LANES // D

NC, NS = 2, 16
NW = NC * NS
PER_W = NUM_IDX // NW
CHUNK = 128
N_CHUNK = PER_W // CHUNK
GROUPS = CHUNK // 16


def kernel(prev_act, tables):
    wide = tables.reshape(NH * VOCAB // PACK, LANES)
    offs = jnp.arange(NH, dtype=jnp.int32) * VOCAB
    g = (prev_act.reshape(BS * NA, NH) + offs[None, :]).reshape(
        NW, N_CHUNK, GROUPS, 16
    )

    mesh = plsc.VectorSubcoreMesh(core_axis_name="c", subcore_axis_name="s")
    cp = pltpu.CompilerParams()
    if "needs_layout_passes" in pltpu.CompilerParams.__dataclass_fields__:
        cp = dataclasses.replace(cp, needs_layout_passes=False)

    @pl.kernel(
        out_type=jax.ShapeDtypeStruct((NUM_IDX, D), jnp.float32),
        mesh=mesh,
        compiler_params=cp,
        scratch_types=[
            pltpu.VMEM((4, GROUPS, 16), jnp.int32),
            pltpu.VMEM((2, CHUNK), jnp.int32),
            pltpu.VMEM((2, CHUNK, LANES), jnp.float32),
            pltpu.VMEM((2, CHUNK, D), jnp.float32),
            pltpu.SemaphoreType.DMA,
            pltpu.SemaphoreType.DMA,
            pltpu.SemaphoreType.DMA,
            pltpu.SemaphoreType.DMA,
            pltpu.SemaphoreType.DMA,
            pltpu.SemaphoreType.DMA,
            pltpu.SemaphoreType.DMA,
            pltpu.SemaphoreType.DMA,
        ],
    )
    def gather_kernel(wide_hbm, g_hbm, out_hbm,
                      g_v, idx4_v, rows_v, out_v,
                      ssem0, ssem1, ssem2, ssem3, gsem0, gsem1, wsem0, wsem1):
        ssems = (ssem0, ssem1, ssem2, ssem3)
        gsems = (gsem0, gsem1)
        wsems = (wsem0, wsem1)
        wid = lax.axis_index("s") * NC + lax.axis_index("c")
        base = wid * PER_W
        iota = lax.iota(jnp.int32, 16)

        def sstart(c, r):
            pltpu.async_copy(g_hbm.at[wid, c], g_v.at[r], ssems[r])

        def swait(r):
            pltpu.make_async_copy(
                g_hbm.at[wid, 0], g_v.at[r], ssems[r]
            ).wait()

        def compute_idx4(r, b):
            @pl.loop(0, GROUPS)
            def _(k):
                idx4_v[b, pl.ds(k * 16, 16)] = g_v[r, k] >> 2

        def gstart(b):
            pltpu.async_copy(wide_hbm.at[idx4_v.at[b]], rows_v.at[b], gsems[b])

        def gwait(b):
            pltpu.make_async_copy(
                wide_hbm.at[idx4_v.at[b]], rows_v.at[b], gsems[b]
            ).wait()

        def extract(r, b):
            @pl.loop(0, GROUPS)
            def _(gi):
                rvec = iota + gi * 16
                s16 = (g_v[r, gi] & 3) * D
                for j in range(D):
                    val = plsc.load_gather(rows_v.at[b], [rvec, s16 + j])
                    plsc.store_scatter(out_v.at[b], [rvec, iota * 0 + j], val)

        def wstart(c, b):
            pltpu.async_copy(
                out_v.at[b], out_hbm.at[pl.ds(base + c * CHUNK, CHUNK)],
                wsems[b]
            )

        def wwait(b):
            pltpu.make_async_copy(
                out_v.at[b], out_hbm.at[pl.ds(base, CHUNK)], wsems[b]
            ).wait()

        for r in range(4):
            sstart(r, r)
        swait(0)
        compute_idx4(0, 0)
        gstart(0)
        swait(1)
        compute_idx4(1, 1)
        gstart(1)
        gwait(0)
        extract(0, 0)
        wstart(0, 0)
        sstart(4, 0)
        swait(2)
        compute_idx4(2, 0)
        gstart(0)
        gwait(1)
        extract(1, 1)
        wstart(1, 1)
        sstart(5, 1)
        swait(3)
        compute_idx4(3, 1)
        gstart(1)
        gwait(0)
        wwait(0)
        extract(2, 0)
        wstart(2, 0)
        sstart(6, 2)
        swait(0)
        compute_idx4(0, 0)
        gstart(0)
        gwait(1)
        wwait(1)
        extract(3, 1)
        wstart(3, 1)
        sstart(7, 3)

        @pl.loop(1, N_CHUNK // 4 - 1)
        def _(q):
            c0 = 4 * q
            for r in range(4):
                c = c0 + r
                swait((r + 1) % 4)
                compute_idx4((r + 1) % 4, (r + 1) % 2)
                gstart((r + 1) % 2)
                gwait(r % 2)
                wwait(r % 2)
                extract(r, r % 2)
                wstart(c, r % 2)
                sstart(c + 4, r)

        ce = N_CHUNK - 4
        for r in range(3):
            c = ce + r
            swait((r + 1) % 4)
            compute_idx4((r + 1) % 4, (r + 1) % 2)
            gstart((r + 1) % 2)
            gwait(r % 2)
            wwait(r % 2)
            extract(r, r % 2)
            wstart(c, r % 2)
        gwait(1)
        wwait(1)
        extract(3, 1)
        wstart(N_CHUNK - 1, 1)
        wwait(0)
        wwait(1)

    out = gather_kernel(wide, g)
    return out.reshape(BS, NA, NH * D)


# speedup vs baseline: 1.6226x; 1.3930x over previous
import dataclasses

import jax
import jax.numpy as jnp
from jax import lax
from jax.experimental import pallas as pl
from jax.experimental.pallas import tpu as pltpu
from jax.experimental.pallas import tpu_sc as plsc

BS, NA, NH = 1024, 16, 26
VOCAB, D = 100000, 32
NUM_IDX = BS * NA * NH

NC, NS = 2, 16
NW = NC * NS
PER_W = NUM_IDX // NW
CHUNK = 128
N_CHUNK = PER_W // CHUNK


def kernel(prev_act, tables):
    flat_tables = tables.reshape(NH * VOCAB, D)
    offs = jnp.arange(NH, dtype=jnp.int32) * VOCAB
    g = (prev_act.reshape(BS * NA, NH) + offs[None, :]).reshape(
        NW, N_CHUNK, CHUNK
    )

    mesh = plsc.VectorSubcoreMesh(core_axis_name="c", subcore_axis_name="s")
    cp = pltpu.CompilerParams(use_tc_tiling_on_sc=False)
    if "needs_layout_passes" in pltpu.CompilerParams.__dataclass_fields__:
        cp = dataclasses.replace(cp, needs_layout_passes=False)

    @pl.kernel(
        out_type=jax.ShapeDtypeStruct((NUM_IDX, D), jnp.float32),
        mesh=mesh,
        compiler_params=cp,
        scratch_types=[
            pltpu.VMEM((4, 1, CHUNK), jnp.int32),
            pltpu.VMEM((2, CHUNK, D), jnp.float32),
            pltpu.SemaphoreType.DMA,
            pltpu.SemaphoreType.DMA,
            pltpu.SemaphoreType.DMA,
            pltpu.SemaphoreType.DMA,
            pltpu.SemaphoreType.DMA,
            pltpu.SemaphoreType.DMA,
            pltpu.SemaphoreType.DMA,
            pltpu.SemaphoreType.DMA,
        ],
    )
    def gather_kernel(table_hbm, g_hbm, out_hbm, idx_v, out_v,
                      isem0, isem1, isem2, isem3, gsem0, gsem1, wsem0, wsem1):
        isems = (isem0, isem1, isem2, isem3)
        gsems = (gsem0, gsem1)
        wsems = (wsem0, wsem1)
        wid = lax.axis_index("s") * NC + lax.axis_index("c")
        base = wid * PER_W

        def istart(c, r):
            pltpu.async_copy(g_hbm.at[wid, c], idx_v.at[r, 0], isems[r])

        def iwait(r):
            pltpu.make_async_copy(
                g_hbm.at[wid, 0], idx_v.at[r, 0], isems[r]
            ).wait()

        def gstart(r, b):
            pltpu.async_copy(
                table_hbm.at[idx_v.at[r, 0]], out_v.at[b], gsems[b]
            )

        def gwait(b):
            pltpu.make_async_copy(
                table_hbm.at[idx_v.at[0, 0]], out_v.at[b], gsems[b]
            ).wait()

        def wstart(c, b):
            pltpu.async_copy(
                out_v.at[b], out_hbm.at[pl.ds(base + c * CHUNK, CHUNK)],
                wsems[b]
            )

        def wwait(b):
            pltpu.make_async_copy(
                out_v.at[b], out_hbm.at[pl.ds(base, CHUNK)], wsems[b]
            ).wait()

        for r in range(4):
            istart(r, r)
        iwait(0)
        gstart(0, 0)
        iwait(1)
        gstart(1, 1)
        gwait(0)
        wstart(0, 0)
        istart(4, 0)
        iwait(2)
        wwait(0)
        gstart(2, 0)
        gwait(1)
        wstart(1, 1)
        istart(5, 1)
        iwait(3)
        wwait(1)
        gstart(3, 1)
        gwait(0)
        wstart(2, 0)
        istart(6, 2)
        iwait(0)
        wwait(0)
        gstart(0, 0)
        gwait(1)
        wstart(3, 1)
        istart(7, 3)

        @pl.loop(1, N_CHUNK // 4 - 1)
        def _(q):
            c0 = 4 * q
            for r in range(4):
                c = c0 + r
                iwait((r + 1) % 4)
                wwait((r + 1) % 2)
                gstart((r + 1) % 4, (r + 1) % 2)
                gwait(r % 2)
                wstart(c, r % 2)
                istart(c + 4, r)

        ce = N_CHUNK - 4
        for r in range(3):
            c = ce + r
            iwait((r + 1) % 4)
            wwait((r + 1) % 2)
            gstart((r + 1) % 4, (r + 1) % 2)
            gwait(r % 2)
            wstart(c, r % 2)
        gwait(1)
        wstart(N_CHUNK - 1, 1)
        wwait(0)
        wwait(1)

    out = gather_kernel(flat_tables, g)
    return out.reshape(BS, NA, NH * D)
